# baseline (speedup 1.0000x reference)
import functools

import jax
import jax.numpy as jnp
from jax.experimental import pallas as pl
from jax.experimental.pallas import tpu as pltpu

_GROUP = 2


def _round_up(v, m):
    return (v + m - 1) // m * m


def _conv3_kernel(w_ref, x_hbm, o_hbm, x_buf, o_buf, in_sem, out_sem,
                  *, rows, group):
    c_id = pl.program_id(0)
    row0 = c_id * rows
    l = x_buf.shape[2]

    if rows >= 6 and (rows - 4) % group == 0:
        in_sizes = [1, 1, 1, 1] + [group] * ((rows - 4) // group)
        out_sizes = [group] * ((rows - 4) // group) + [1, 1, 1, 1]
    elif rows >= 4 and (rows - 2) % group == 0:
        in_sizes = [1, 1] + [group] * ((rows - 2) // group)
        out_sizes = [group] * ((rows - 2) // group) + [1, 1]
    else:
        in_sizes = [1] * rows
        out_sizes = [1] * rows
    in_starts = [sum(in_sizes[:i]) for i in range(len(in_sizes))]
    out_starts = [sum(out_sizes[:i]) for i in range(len(out_sizes))]
    in_wait_at = {s: i for i, s in enumerate(in_starts)}
    out_issue_at = {s + sz - 1: (s, sz) for s, sz in zip(out_starts, out_sizes)}

    in_copies = []
    for s, sz in zip(in_starts, in_sizes):
        cp = pltpu.make_async_copy(
            x_hbm.at[pl.ds(row0 + s, sz)], x_buf.at[pl.ds(s, sz)], in_sem)
        cp.start()
        in_copies.append(cp)

    out_copies = []
    for r in range(rows):
        if r in in_wait_at:
            in_copies[in_wait_at[r]].wait()
        xb = x_buf[r].astype(jnp.bfloat16)
        c = xb.shape[0]
        zero_col = jnp.zeros((c, 1), jnp.bfloat16)
        x3 = jnp.concatenate(
            [jnp.concatenate([zero_col, xb[:, : l - 1]], axis=1),
             xb,
             jnp.concatenate([xb[:, 1:], zero_col], axis=1)], axis=0)
        acc = jnp.dot(w_ref[...], x3, preferred_element_type=jnp.float32)
        o_buf[r] = jnp.maximum(acc, 0.0)
        if r in out_issue_at:
            s, sz = out_issue_at[r]
            cp = pltpu.make_async_copy(
                o_buf.at[pl.ds(s, sz)], o_hbm.at[pl.ds(row0 + s, sz)], out_sem)
            cp.start()
            out_copies.append(cp)
    for cp in out_copies:
        cp.wait()


@jax.jit
def kernel(x, weight):
    N, C_in, L = x.shape
    C_out, C_in_w, K = weight.shape
    assert C_in_w == C_in and K == 3
    L_out = L

    C_in_pad = _round_up(C_in, 8)
    C_out_pad = _round_up(C_out, 8)
    L_pad = _round_up(L, 128)
    xp = jnp.pad(x, ((0, 0), (0, C_in_pad - C_in), (0, L_pad - L)))
    w3 = jnp.transpose(weight, (2, 0, 1)).astype(jnp.bfloat16)
    w3 = jnp.pad(w3, ((0, 0), (0, C_out_pad - C_out), (0, C_in_pad - C_in)))
    w3 = jnp.transpose(w3, (1, 0, 2)).reshape(C_out_pad, K * C_in_pad)

    rows = N // 2
    out = pl.pallas_call(
        functools.partial(_conv3_kernel, rows=rows, group=_GROUP),
        out_shape=jax.ShapeDtypeStruct((N, C_out_pad, L_pad), x.dtype),
        grid=(2,),
        in_specs=[
            pl.BlockSpec((C_out_pad, K * C_in_pad), lambda c: (0, 0)),
            pl.BlockSpec(memory_space=pltpu.MemorySpace.HBM),
        ],
        out_specs=pl.BlockSpec(memory_space=pltpu.MemorySpace.HBM),
        scratch_shapes=[
            pltpu.VMEM((rows, C_in_pad, L_pad), jnp.float32),
            pltpu.VMEM((rows, C_out_pad, L_pad), jnp.float32),
            pltpu.SemaphoreType.DMA,
            pltpu.SemaphoreType.DMA,
        ],
        compiler_params=pltpu.CompilerParams(
            dimension_semantics=("parallel",),
            vmem_limit_bytes=60 * 1024 * 1024,
        ),
    )(w3, xp)
    if C_out_pad != C_out or L_pad != L_out:
        out = out[:, :C_out, :L_out]
    return out


# speedup vs baseline: 1.0218x; 1.0218x over previous
import functools

import jax
import jax.numpy as jnp
from jax.experimental import pallas as pl
from jax.experimental.pallas import tpu as pltpu

_GROUP = 2


def _round_up(v, m):
    return (v + m - 1) // m * m


def _conv3_kernel(w_ref, x_hbm, o_hbm, x_buf, o_buf, in_sem, out_sem,
                  *, rows, group):
    c_id = pl.program_id(0)
    row0 = c_id * rows
    l = x_buf.shape[2]

    if rows >= 4 and (rows - 2) % group == 0:
        in_sizes = [1, 1] + [group] * ((rows - 2) // group)
        out_sizes = [group] * ((rows - 2) // group) + [1, 1]
    else:
        in_sizes = [1] * rows
        out_sizes = [1] * rows
    in_starts = [sum(in_sizes[:i]) for i in range(len(in_sizes))]
    out_starts = [sum(out_sizes[:i]) for i in range(len(out_sizes))]
    in_wait_at = {s: i for i, s in enumerate(in_starts)}
    out_issue_at = {s + sz - 1: (s, sz) for s, sz in zip(out_starts, out_sizes)}

    in_copies = []
    for s, sz in zip(in_starts, in_sizes):
        cp = pltpu.make_async_copy(
            x_hbm.at[pl.ds(row0 + s, sz)], x_buf.at[pl.ds(s, sz)], in_sem)
        cp.start()
        in_copies.append(cp)

    out_copies = []
    for r in range(rows):
        if r in in_wait_at:
            in_copies[in_wait_at[r]].wait()
        xb = x_buf[r].astype(jnp.bfloat16)
        c = xb.shape[0]
        zero_col = jnp.zeros((c, 1), jnp.bfloat16)
        x3 = jnp.concatenate(
            [jnp.concatenate([zero_col, xb[:, : l - 1]], axis=1),
             xb,
             jnp.concatenate([xb[:, 1:], zero_col], axis=1)], axis=0)
        acc = jnp.dot(w_ref[...], x3, preferred_element_type=jnp.float32)
        o_buf[r] = jnp.maximum(acc, 0.0)
        if r in out_issue_at:
            s, sz = out_issue_at[r]
            cp = pltpu.make_async_copy(
                o_buf.at[pl.ds(s, sz)], o_hbm.at[pl.ds(row0 + s, sz)], out_sem)
            cp.start()
            out_copies.append(cp)
    for cp in out_copies:
        cp.wait()


@jax.jit
def kernel(x, weight):
    N, C_in, L = x.shape
    C_out, C_in_w, K = weight.shape
    assert C_in_w == C_in and K == 3
    L_out = L

    C_in_pad = _round_up(C_in, 8)
    C_out_pad = _round_up(C_out, 8)
    L_pad = _round_up(L, 128)
    xp = jnp.pad(x, ((0, 0), (0, C_in_pad - C_in), (0, L_pad - L)))
    w3 = jnp.transpose(weight, (2, 0, 1)).astype(jnp.bfloat16)
    w3 = jnp.pad(w3, ((0, 0), (0, C_out_pad - C_out), (0, C_in_pad - C_in)))
    w3 = jnp.transpose(w3, (1, 0, 2)).reshape(C_out_pad, K * C_in_pad)

    rows = N // 2
    out = pl.pallas_call(
        functools.partial(_conv3_kernel, rows=rows, group=_GROUP),
        out_shape=jax.ShapeDtypeStruct((N, C_out_pad, L_pad), x.dtype),
        grid=(2,),
        in_specs=[
            pl.BlockSpec((C_out_pad, K * C_in_pad), lambda c: (0, 0)),
            pl.BlockSpec(memory_space=pltpu.MemorySpace.HBM),
        ],
        out_specs=pl.BlockSpec(memory_space=pltpu.MemorySpace.HBM),
        scratch_shapes=[
            pltpu.VMEM((rows, C_in_pad, L_pad), jnp.float32),
            pltpu.VMEM((rows, C_out_pad, L_pad), jnp.float32),
            pltpu.SemaphoreType.DMA,
            pltpu.SemaphoreType.DMA,
        ],
        compiler_params=pltpu.CompilerParams(
            dimension_semantics=("parallel",),
            vmem_limit_bytes=60 * 1024 * 1024,
        ),
    )(w3, xp)
    if C_out_pad != C_out or L_pad != L_out:
        out = out[:, :C_out, :L_out]
    return out
